# Initial kernel scaffold; baseline (speedup 1.0000x reference)
import jax
import jax.numpy as jnp
from jax.experimental import pallas as pl


def kernel(hidden_states, gate_w, w_gate, w_up, w_down):
    raise NotImplementedError("write your pallas kernel here")


# speedup vs baseline: 1.2683x; 1.2683x over previous
import functools

import jax
import jax.numpy as jnp
from jax import lax
from jax.experimental import pallas as pl
from jax.experimental.pallas import tpu as pltpu
from jax.experimental.pallas import tpu_sc as plsc

T, D = 2048, 2048
E, K, F = 8, 2, 768

BT = 128
NBS = T * K // BT + 7
NS = NBS * BT

NC, NSUB = 2, 16
NW = NC * NSUB
TPT = T // NW
CH = 16
NCH = TPT // CH


def _router_body(x_ref, gate_ref, posa_ref, posb_ref, rw1_ref, rw2_ref,
                 map_ref):
    logits = lax.dot_general(
        x_ref[...].astype(jnp.bfloat16), gate_ref[...].astype(jnp.bfloat16),
        (((1,), (1,)), ((), ())), preferred_element_type=jnp.float32)
    iota = lax.broadcasted_iota(jnp.int32, (T, E), 1)
    m1 = jnp.max(logits, axis=1, keepdims=True)
    a1 = jnp.min(jnp.where(logits == m1, iota, E), axis=1, keepdims=True)
    masked = jnp.where(iota == a1, -1e30, logits)
    m2 = jnp.max(masked, axis=1, keepdims=True)
    a2 = jnp.min(jnp.where(masked == m2, iota, E), axis=1, keepdims=True)
    rw1 = jax.nn.sigmoid(m1 - m2)
    rw1_ref[...] = rw1
    rw2_ref[...] = 1.0 - rw1

    oha = (iota == a1).astype(jnp.int32)
    ohb = (iota == a2).astype(jnp.int32)

    def _cumsum0(v):
        sh = 1
        while sh < T:
            v = v + jnp.concatenate(
                [jnp.zeros((sh, E), v.dtype), v[:T - sh]], axis=0)
            sh *= 2
        return v

    ca = _cumsum0(oha)
    cb = _cumsum0(ohb) + ca[T - 1:T, :]
    counts = cb[T - 1:T, :]
    blk = (counts + BT - 1) // BT
    ei = lax.broadcasted_iota(jnp.int32, (E, E), 0)
    ej = lax.broadcasted_iota(jnp.int32, (E, E), 1)
    ltri = (ei <= ej).astype(jnp.float32)
    blk_cum = lax.dot_general(blk.astype(jnp.float32), ltri,
                              (((1,), (0,)), ((), ())),
                              preferred_element_type=jnp.float32)
    blk_start = blk_cum.astype(jnp.int32) - blk
    base = BT * blk_start
    posa_ref[...] = jnp.sum(oha * (base + ca - 1), axis=1, keepdims=True)
    posb_ref[...] = jnp.sum(ohb * (base + cb - 1), axis=1, keepdims=True)

    biota = lax.broadcasted_iota(jnp.int32, (NBS, E), 0)
    bs = jnp.broadcast_to(blk_start, (NBS, E))
    mp = jnp.sum((bs <= biota).astype(jnp.int32), axis=1, keepdims=True) - 1
    map_ref[...] = jnp.clip(mp, 0, E - 1)


def _router(x, gate_w):
    return pl.pallas_call(
        _router_body,
        out_shape=(
            jax.ShapeDtypeStruct((T, 1), jnp.int32),
            jax.ShapeDtypeStruct((T, 1), jnp.int32),
            jax.ShapeDtypeStruct((T, 1), jnp.float32),
            jax.ShapeDtypeStruct((T, 1), jnp.float32),
            jax.ShapeDtypeStruct((NBS, 1), jnp.int32),
        ),
    )(x, gate_w)


def _scatter_body(x_hbm, pos_hbm, rwx_hbm, xs_hbm, ws_hbm,
                  idx_v, rows_v, rwrows_v, sem):
    c = lax.axis_index("c")
    s = lax.axis_index("s")
    tok0 = (s * NC + c) * TPT
    for ch in range(NCH):
        pltpu.sync_copy(pos_hbm.at[pl.ds(tok0 + CH * ch, CH)], idx_v.at[ch])
        pltpu.sync_copy(pos_hbm.at[pl.ds(T + tok0 + CH * ch, CH)],
                        idx_v.at[NCH + ch])
    for ch in range(NCH):
        pltpu.sync_copy(x_hbm.at[pl.ds(tok0 + CH * ch, CH)], rows_v)
        pltpu.sync_copy(rwx_hbm.at[pl.ds(tok0 + CH * ch, CH)], rwrows_v.at[0])
        pltpu.sync_copy(rwx_hbm.at[pl.ds(T + tok0 + CH * ch, CH)],
                        rwrows_v.at[1])
        cps = [
            pltpu.async_copy(rows_v, xs_hbm.at[idx_v.at[ch]], sem),
            pltpu.async_copy(rows_v, xs_hbm.at[idx_v.at[NCH + ch]], sem),
            pltpu.async_copy(rwrows_v.at[0], ws_hbm.at[idx_v.at[ch]], sem),
            pltpu.async_copy(rwrows_v.at[1], ws_hbm.at[idx_v.at[NCH + ch]],
                             sem),
        ]
        for cp in cps:
            cp.wait()


_scatter = functools.partial(
    pl.kernel,
    out_type=(
        jax.ShapeDtypeStruct((NS, D), jnp.float32),
        jax.ShapeDtypeStruct((NS, 128), jnp.float32),
    ),
    mesh=plsc.VectorSubcoreMesh(core_axis_name="c", subcore_axis_name="s",
                                num_cores=NC, num_subcores=NSUB),
    scratch_types=[
        pltpu.VMEM((2 * NCH, CH), jnp.int32),
        pltpu.VMEM((CH, D), jnp.float32),
        pltpu.VMEM((2, CH, 128), jnp.float32),
        pltpu.SemaphoreType.DMA,
    ],
)(_scatter_body)


def _ffn_body(map_ref, xs_ref, wg_ref, wu_ref, wd_ref, ws_ref, ys_ref):
    del map_ref
    x16 = xs_ref[...].astype(jnp.bfloat16)
    wg16 = wg_ref[0].astype(jnp.bfloat16)
    wu16 = wu_ref[0].astype(jnp.bfloat16)
    g = lax.dot_general(x16, wg16, (((1,), (1,)), ((), ())),
                        preferred_element_type=jnp.float32)
    u = lax.dot_general(x16, wu16, (((1,), (1,)), ((), ())),
                        preferred_element_type=jnp.float32)
    h16 = ((g * jax.nn.sigmoid(g)) * u).astype(jnp.bfloat16)
    wd16 = wd_ref[0].astype(jnp.bfloat16)
    y = lax.dot_general(h16, wd16, (((1,), (1,)), ((), ())),
                        preferred_element_type=jnp.float32)
    ys_ref[...] = y * ws_ref[...][:, 0:1]


def _ffn(mp, xs, w_gate, w_up, w_down, ws):
    grid_spec = pltpu.PrefetchScalarGridSpec(
        num_scalar_prefetch=1,
        grid=(NBS,),
        in_specs=[
            pl.BlockSpec((BT, D), lambda b, m: (b, 0)),
            pl.BlockSpec((1, F, D), lambda b, m: (m[b], 0, 0)),
            pl.BlockSpec((1, F, D), lambda b, m: (m[b], 0, 0)),
            pl.BlockSpec((1, D, F), lambda b, m: (m[b], 0, 0)),
            pl.BlockSpec((BT, 128), lambda b, m: (b, 0)),
        ],
        out_specs=pl.BlockSpec((BT, D), lambda b, m: (b, 0)),
    )
    return pl.pallas_call(
        _ffn_body,
        grid_spec=grid_spec,
        out_shape=jax.ShapeDtypeStruct((NS, D), jnp.float32),
    )(mp, xs, w_gate, w_up, w_down, ws)


def _combine_body(ys_hbm, pos_hbm, out_hbm, idx_v, bufa, bufb, obuf, sem):
    c = lax.axis_index("c")
    s = lax.axis_index("s")
    tok0 = (s * NC + c) * TPT
    for ch in range(NCH):
        pltpu.sync_copy(pos_hbm.at[pl.ds(tok0 + CH * ch, CH)], idx_v.at[ch])
        pltpu.sync_copy(pos_hbm.at[pl.ds(T + tok0 + CH * ch, CH)],
                        idx_v.at[NCH + ch])
    for ch in range(NCH):
        cpa = pltpu.async_copy(ys_hbm.at[idx_v.at[ch]], bufa, sem)
        cpb = pltpu.async_copy(ys_hbm.at[idx_v.at[NCH + ch]], bufb, sem)
        cpa.wait()
        cpb.wait()

        def _add(j, carry):
            for i in range(CH):
                obuf[i, pl.ds(j * 16, 16)] = (bufa[i, pl.ds(j * 16, 16)]
                                              + bufb[i, pl.ds(j * 16, 16)])
            return carry

        lax.fori_loop(0, D // 16, _add, 0)
        pltpu.sync_copy(obuf, out_hbm.at[pl.ds(tok0 + CH * ch, CH)])


_combine = functools.partial(
    pl.kernel,
    out_type=jax.ShapeDtypeStruct((T, D), jnp.float32),
    mesh=plsc.VectorSubcoreMesh(core_axis_name="c", subcore_axis_name="s",
                                num_cores=NC, num_subcores=NSUB),
    scratch_types=[
        pltpu.VMEM((2 * NCH, CH), jnp.int32),
        pltpu.VMEM((CH, D), jnp.float32),
        pltpu.VMEM((CH, D), jnp.float32),
        pltpu.VMEM((CH, D), jnp.float32),
        pltpu.SemaphoreType.DMA,
    ],
)(_combine_body)


def kernel(hidden_states, gate_w, w_gate, w_up, w_down):
    b, s_, d_ = hidden_states.shape
    x = hidden_states.reshape(T, D)
    posa, posb, rw1, rw2, mp = _router(x, gate_w)
    pos = jnp.concatenate([posa.reshape(T), posb.reshape(T)])
    rw = jnp.concatenate([rw1.reshape(T), rw2.reshape(T)])
    rwx = jnp.broadcast_to(rw[:, None], (T * K, 128))
    xs, ws = _scatter(x, pos, rwx)
    ys = _ffn(mp.reshape(NBS), xs, w_gate, w_up, w_down, ws)
    out = _combine(ys, pos)
    return out.reshape(b, s_, d_)


# speedup vs baseline: 1.6126x; 1.2715x over previous
import functools

import jax
import jax.numpy as jnp
from jax import lax
from jax.experimental import pallas as pl
from jax.experimental.pallas import tpu as pltpu
from jax.experimental.pallas import tpu_sc as plsc

T, D = 2048, 2048
E, K, F = 8, 2, 768

BT = 256
NBS = T * K // BT + 7
NS = NBS * BT

NC, NSUB = 2, 16
NW = NC * NSUB
TPT = T // NW
CH = 16
NCH = TPT // CH


def _router_body(x_ref, gate_ref, posa_ref, posb_ref, rw1_ref, rw2_ref,
                 map_ref):
    logits = lax.dot_general(
        x_ref[...].astype(jnp.bfloat16), gate_ref[...].astype(jnp.bfloat16),
        (((1,), (1,)), ((), ())), preferred_element_type=jnp.float32)
    iota = lax.broadcasted_iota(jnp.int32, (T, E), 1)
    m1 = jnp.max(logits, axis=1, keepdims=True)
    a1 = jnp.min(jnp.where(logits == m1, iota, E), axis=1, keepdims=True)
    masked = jnp.where(iota == a1, -1e30, logits)
    m2 = jnp.max(masked, axis=1, keepdims=True)
    a2 = jnp.min(jnp.where(masked == m2, iota, E), axis=1, keepdims=True)
    rw1 = jax.nn.sigmoid(m1 - m2)
    rw1_ref[...] = rw1
    rw2_ref[...] = 1.0 - rw1

    oha = (iota == a1).astype(jnp.int32)
    ohb = (iota == a2).astype(jnp.int32)

    def _cumsum0(v):
        sh = 1
        while sh < T:
            v = v + jnp.concatenate(
                [jnp.zeros((sh, E), v.dtype), v[:T - sh]], axis=0)
            sh *= 2
        return v

    ca = _cumsum0(oha)
    cb = _cumsum0(ohb) + ca[T - 1:T, :]
    counts = cb[T - 1:T, :]
    blk = (counts + BT - 1) // BT
    ei = lax.broadcasted_iota(jnp.int32, (E, E), 0)
    ej = lax.broadcasted_iota(jnp.int32, (E, E), 1)
    ltri = (ei <= ej).astype(jnp.float32)
    blk_cum = lax.dot_general(blk.astype(jnp.float32), ltri,
                              (((1,), (0,)), ((), ())),
                              preferred_element_type=jnp.float32)
    blk_start = blk_cum.astype(jnp.int32) - blk
    base = BT * blk_start
    posa_ref[...] = jnp.sum(oha * (base + ca - 1), axis=1, keepdims=True)
    posb_ref[...] = jnp.sum(ohb * (base + cb - 1), axis=1, keepdims=True)

    biota = lax.broadcasted_iota(jnp.int32, (NBS, E), 0)
    bs = jnp.broadcast_to(blk_start, (NBS, E))
    mp = jnp.sum((bs <= biota).astype(jnp.int32), axis=1, keepdims=True) - 1
    map_ref[...] = jnp.clip(mp, 0, E - 1)


def _router(x, gate_w):
    return pl.pallas_call(
        _router_body,
        out_shape=(
            jax.ShapeDtypeStruct((T, 1), jnp.int32),
            jax.ShapeDtypeStruct((T, 1), jnp.int32),
            jax.ShapeDtypeStruct((T, 1), jnp.float32),
            jax.ShapeDtypeStruct((T, 1), jnp.float32),
            jax.ShapeDtypeStruct((NBS, 1), jnp.int32),
        ),
    )(x, gate_w)


def _scatter_body(x_hbm, pos_hbm, rwx_hbm, xs_hbm, ws_hbm,
                  idx_v, rows_v, rwrows_v, sem):
    c = lax.axis_index("c")
    s = lax.axis_index("s")
    tok0 = (s * NC + c) * TPT
    for ch in range(NCH):
        pltpu.sync_copy(pos_hbm.at[pl.ds(tok0 + CH * ch, CH)], idx_v.at[ch])
        pltpu.sync_copy(pos_hbm.at[pl.ds(T + tok0 + CH * ch, CH)],
                        idx_v.at[NCH + ch])
    for ch in range(NCH):
        pltpu.sync_copy(x_hbm.at[pl.ds(tok0 + CH * ch, CH)], rows_v)
        pltpu.sync_copy(rwx_hbm.at[pl.ds(tok0 + CH * ch, CH)], rwrows_v.at[0])
        pltpu.sync_copy(rwx_hbm.at[pl.ds(T + tok0 + CH * ch, CH)],
                        rwrows_v.at[1])
        cps = [
            pltpu.async_copy(rows_v, xs_hbm.at[idx_v.at[ch]], sem),
            pltpu.async_copy(rows_v, xs_hbm.at[idx_v.at[NCH + ch]], sem),
            pltpu.async_copy(rwrows_v.at[0], ws_hbm.at[idx_v.at[ch]], sem),
            pltpu.async_copy(rwrows_v.at[1], ws_hbm.at[idx_v.at[NCH + ch]],
                             sem),
        ]
        for cp in cps:
            cp.wait()


_scatter = functools.partial(
    pl.kernel,
    out_type=(
        jax.ShapeDtypeStruct((NS, D), jnp.float32),
        jax.ShapeDtypeStruct((NS, 128), jnp.float32),
    ),
    mesh=plsc.VectorSubcoreMesh(core_axis_name="c", subcore_axis_name="s",
                                num_cores=NC, num_subcores=NSUB),
    scratch_types=[
        pltpu.VMEM((2 * NCH, CH), jnp.int32),
        pltpu.VMEM((CH, D), jnp.float32),
        pltpu.VMEM((2, CH, 128), jnp.float32),
        pltpu.SemaphoreType.DMA,
    ],
)(_scatter_body)


def _ffn_body(map_ref, xs_ref, wg_ref, wu_ref, wd_ref, ws_ref, ys_ref):
    del map_ref
    x16 = xs_ref[...].astype(jnp.bfloat16)
    wg16 = wg_ref[0].astype(jnp.bfloat16)
    wu16 = wu_ref[0].astype(jnp.bfloat16)
    g = lax.dot_general(x16, wg16, (((1,), (1,)), ((), ())),
                        preferred_element_type=jnp.float32)
    u = lax.dot_general(x16, wu16, (((1,), (1,)), ((), ())),
                        preferred_element_type=jnp.float32)
    h16 = ((g * jax.nn.sigmoid(g)) * u).astype(jnp.bfloat16)
    wd16 = wd_ref[0].astype(jnp.bfloat16)
    y = lax.dot_general(h16, wd16, (((1,), (1,)), ((), ())),
                        preferred_element_type=jnp.float32)
    ys_ref[...] = y * ws_ref[...][:, 0:1]


def _ffn(mp, xs, w_gate, w_up, w_down, ws):
    grid_spec = pltpu.PrefetchScalarGridSpec(
        num_scalar_prefetch=1,
        grid=(NBS,),
        in_specs=[
            pl.BlockSpec((BT, D), lambda b, m: (b, 0)),
            pl.BlockSpec((1, F, D), lambda b, m: (m[b], 0, 0)),
            pl.BlockSpec((1, F, D), lambda b, m: (m[b], 0, 0)),
            pl.BlockSpec((1, D, F), lambda b, m: (m[b], 0, 0)),
            pl.BlockSpec((BT, 128), lambda b, m: (b, 0)),
        ],
        out_specs=pl.BlockSpec((BT, D), lambda b, m: (b, 0)),
    )
    return pl.pallas_call(
        _ffn_body,
        grid_spec=grid_spec,
        out_shape=jax.ShapeDtypeStruct((NS, D), jnp.float32),
    )(mp, xs, w_gate, w_up, w_down, ws)


def _combine_body(ys_hbm, pos_hbm, out_hbm, idx_v, bufa, bufb, obuf, sem):
    c = lax.axis_index("c")
    s = lax.axis_index("s")
    tok0 = (s * NC + c) * TPT
    for ch in range(NCH):
        pltpu.sync_copy(pos_hbm.at[pl.ds(tok0 + CH * ch, CH)], idx_v.at[ch])
        pltpu.sync_copy(pos_hbm.at[pl.ds(T + tok0 + CH * ch, CH)],
                        idx_v.at[NCH + ch])
    for ch in range(NCH):
        cpa = pltpu.async_copy(ys_hbm.at[idx_v.at[ch]], bufa, sem)
        cpb = pltpu.async_copy(ys_hbm.at[idx_v.at[NCH + ch]], bufb, sem)
        cpa.wait()
        cpb.wait()

        def _add(j, carry):
            for i in range(CH):
                obuf[i, pl.ds(j * 16, 16)] = (bufa[i, pl.ds(j * 16, 16)]
                                              + bufb[i, pl.ds(j * 16, 16)])
            return carry

        lax.fori_loop(0, D // 16, _add, 0)
        pltpu.sync_copy(obuf, out_hbm.at[pl.ds(tok0 + CH * ch, CH)])


_combine = functools.partial(
    pl.kernel,
    out_type=jax.ShapeDtypeStruct((T, D), jnp.float32),
    mesh=plsc.VectorSubcoreMesh(core_axis_name="c", subcore_axis_name="s",
                                num_cores=NC, num_subcores=NSUB),
    scratch_types=[
        pltpu.VMEM((2 * NCH, CH), jnp.int32),
        pltpu.VMEM((CH, D), jnp.float32),
        pltpu.VMEM((CH, D), jnp.float32),
        pltpu.VMEM((CH, D), jnp.float32),
        pltpu.SemaphoreType.DMA,
    ],
)(_combine_body)


def kernel(hidden_states, gate_w, w_gate, w_up, w_down):
    b, s_, d_ = hidden_states.shape
    x = hidden_states.reshape(T, D)
    posa, posb, rw1, rw2, mp = _router(x, gate_w)
    pos = jnp.concatenate([posa.reshape(T), posb.reshape(T)])
    rw = jnp.concatenate([rw1.reshape(T), rw2.reshape(T)])
    rwx = jnp.broadcast_to(rw[:, None], (T * K, 128))
    xs, ws = _scatter(x, pos, rwx)
    ys = _ffn(mp.reshape(NBS), xs, w_gate, w_up, w_down, ws)
    out = _combine(ys, pos)
    return out.reshape(b, s_, d_)
